# baseline (speedup 1.0000x reference)
import functools

import jax
import jax.numpy as jnp
from jax import lax
from jax.experimental import pallas as pl
from jax.experimental.pallas import tpu as pltpu
from jax.experimental.pallas import tpu_sc as plsc

NSTREAM = 4


@functools.cache
def _make(n_cols: int, n_tokens: int, dim: int):
    info = plsc.get_sparse_core_info()
    nw = info.num_cores * info.num_subcores
    tpw = n_tokens // nw
    seg = tpw // NSTREAM
    npair = n_cols // 2
    mesh = plsc.VectorSubcoreMesh(core_axis_name="c", subcore_axis_name="s")
    lanes = info.num_lanes
    ngrp = tpw // lanes

    @functools.partial(
        pl.kernel,
        mesh=mesh,
        out_type=jax.ShapeDtypeStruct((n_cols, dim, n_tokens), jnp.float32),
        scratch_types=[
            pltpu.VMEM((n_cols, tpw), jnp.int32),
            pltpu.VMEM((tpw, dim), jnp.float32),
            pltpu.VMEM((tpw, dim), jnp.float32),
            pltpu.VMEM((dim, tpw), jnp.float32),
            pltpu.VMEM((dim, tpw), jnp.float32),
            pltpu.SemaphoreType.DMA,
            pltpu.SemaphoreType.DMA,
            pltpu.SemaphoreType.DMA,
            pltpu.SemaphoreType.DMA,
        ],
        compiler_params=pltpu.CompilerParams(
            use_tc_tiling_on_sc=False, needs_layout_passes=False
        ),
    )
    def emb(xt_hbm, table_hbm, out_hbm, idx_t, rows0, rows1, tb0, tb1,
            gsem0, gsem1, wsem0, wsem1):
        wid = lax.axis_index("s") * info.num_cores + lax.axis_index("c")
        i0 = wid * tpw
        pltpu.sync_copy(xt_hbm.at[:, pl.ds(i0, tpw)], idx_t)
        rows = (rows0, rows1)
        tbufs = (tb0, tb1)
        gsems = (gsem0, gsem1)
        wsems = (wsem0, wsem1)

        def gather_descs(j, p):
            return [
                pltpu.make_async_copy(
                    table_hbm.at[idx_t.at[j, pl.ds(q * seg, seg)]],
                    rows[p].at[pl.ds(q * seg, seg)],
                    gsems[p],
                )
                for q in range(NSTREAM)
            ]

        def wb_desc(j, p):
            return pltpu.make_async_copy(
                tbufs[p], out_hbm.at[j, :, pl.ds(i0, tpw)], wsems[p]
            )

        ivecs = [lax.iota(jnp.int32, lanes) + g * lanes for g in range(ngrp)]

        def transpose(p):
            src, dst = rows[p], tbufs[p]

            def tbody(d, carry):
                dvec = jnp.full((lanes,), d, jnp.int32)
                for g in range(ngrp):
                    v = plsc.load_gather(src, [ivecs[g], dvec])
                    dst[d, pl.ds(g * lanes, lanes)] = v
                return carry

            lax.fori_loop(0, dim, tbody, 0)

        for d in gather_descs(0, 0):
            d.start()
        for d in gather_descs(1, 1):
            d.start()

        def body(jj, carry):
            for p in range(2):
                j = 2 * jj + p
                for desc in gather_descs(j, p):
                    desc.wait()

                @pl.when(jj > 0)
                def _():
                    wb_desc(j - 2, p).wait()

                transpose(p)
                wb_desc(j, p).start()

                @pl.when(jj + 1 < npair)
                def _():
                    for desc in gather_descs(j + 2, p):
                        desc.start()

            return carry

        lax.fori_loop(0, npair, body, 0)
        wb_desc(n_cols - 2, 0).wait()
        wb_desc(n_cols - 1, 1).wait()

    return emb


def kernel(x, weights):
    b, s = x.shape
    dim = weights.shape[1]
    xt = jnp.transpose(x.astype(jnp.int32))
    out_t = _make(s, b, dim)(xt, weights)
    return jnp.transpose(out_t, (2, 0, 1))


# speedup vs baseline: 1.1610x; 1.1610x over previous
import functools

import jax
import jax.numpy as jnp
from jax import lax
from jax.experimental import pallas as pl
from jax.experimental.pallas import tpu as pltpu
from jax.experimental.pallas import tpu_sc as plsc

NSTREAM = 4


@functools.cache
def _make(n_cols: int, n_tokens: int, dim: int):
    info = plsc.get_sparse_core_info()
    nw = info.num_cores * info.num_subcores
    tpw = n_tokens // nw
    seg = tpw // NSTREAM
    npair = n_cols // 2
    mesh = plsc.VectorSubcoreMesh(core_axis_name="c", subcore_axis_name="s")
    lanes = info.num_lanes
    ngrp = tpw // lanes

    @functools.partial(
        pl.kernel,
        mesh=mesh,
        out_type=jax.ShapeDtypeStruct((n_cols, dim, n_tokens), jnp.float32),
        scratch_types=[
            pltpu.VMEM((n_cols, tpw), jnp.int32),
            pltpu.VMEM((tpw, dim), jnp.float32),
            pltpu.VMEM((tpw, dim), jnp.float32),
            pltpu.VMEM((dim, tpw), jnp.float32),
            pltpu.VMEM((dim, tpw), jnp.float32),
            pltpu.SemaphoreType.DMA,
            pltpu.SemaphoreType.DMA,
            pltpu.SemaphoreType.DMA,
            pltpu.SemaphoreType.DMA,
        ],
        compiler_params=pltpu.CompilerParams(
            use_tc_tiling_on_sc=False, needs_layout_passes=False
        ),
    )
    def emb(xt_hbm, table_hbm, out_hbm, idx_t, rows0, rows1, tb0, tb1,
            gsem0, gsem1, wsem0, wsem1):
        wid = lax.axis_index("s") * info.num_cores + lax.axis_index("c")
        i0 = wid * tpw
        pltpu.sync_copy(xt_hbm.at[:, pl.ds(i0, tpw)], idx_t)
        rows = (rows0, rows1)
        tbufs = (tb0, tb1)
        gsems = (gsem0, gsem1)
        wsems = (wsem0, wsem1)

        def gather_descs(j, p):
            return [
                pltpu.make_async_copy(
                    table_hbm.at[idx_t.at[j, pl.ds(q * seg, seg)]],
                    rows[p].at[pl.ds(q * seg, seg)],
                    gsems[p],
                )
                for q in range(NSTREAM)
            ]

        def wb_desc(j, p):
            return pltpu.make_async_copy(
                tbufs[p], out_hbm.at[j, :, pl.ds(i0, tpw)], wsems[p]
            )

        ivecs = [lax.iota(jnp.int32, lanes) + g * lanes for g in range(ngrp)]

        def transpose(p):
            src, dst = rows[p], tbufs[p]

            @plsc.parallel_loop(0, dim, 1, unroll=2)
            def tbody(d):
                dvec = jnp.full((lanes,), d, jnp.int32)
                for b in range(0, ngrp, 8):
                    vs = [
                        plsc.load_gather(src, [ivecs[g], dvec])
                        for g in range(b, b + 8)
                    ]
                    for g, v in zip(range(b, b + 8), vs):
                        dst[d, pl.ds(g * lanes, lanes)] = v

        for d in gather_descs(0, 0):
            d.start()
        for d in gather_descs(1, 1):
            d.start()

        def body(jj, carry):
            for p in range(2):
                j = 2 * jj + p
                for desc in gather_descs(j, p):
                    desc.wait()

                @pl.when(jj > 0)
                def _():
                    wb_desc(j - 2, p).wait()

                transpose(p)
                wb_desc(j, p).start()

                @pl.when(jj + 1 < npair)
                def _():
                    for desc in gather_descs(j + 2, p):
                        desc.start()

            return carry

        lax.fori_loop(0, npair, body, 0)
        wb_desc(n_cols - 2, 0).wait()
        wb_desc(n_cols - 1, 1).wait()

    return emb


def kernel(x, weights):
    b, s = x.shape
    dim = weights.shape[1]
    xt = jnp.transpose(x.astype(jnp.int32))
    out_t = _make(s, b, dim)(xt, weights)
    return jnp.transpose(out_t, (2, 0, 1))


# speedup vs baseline: 1.7691x; 1.5238x over previous
import functools

import jax
import jax.numpy as jnp
from jax import lax
from jax.experimental import pallas as pl
from jax.experimental.pallas import tpu as pltpu
from jax.experimental.pallas import tpu_sc as plsc

NSTREAM = 4


@functools.cache
def _make(n_cols: int, n_tokens: int, dim: int):
    info = plsc.get_sparse_core_info()
    nw = info.num_cores * info.num_subcores
    tpw = n_tokens // nw
    seg = tpw // NSTREAM
    npair = n_cols // 2
    mesh = plsc.VectorSubcoreMesh(core_axis_name="c", subcore_axis_name="s")
    lanes = info.num_lanes
    tpad = tpw + 1

    @functools.partial(
        pl.kernel,
        mesh=mesh,
        out_type=jax.ShapeDtypeStruct((n_cols, dim, n_tokens), jnp.float32),
        scratch_types=[
            pltpu.VMEM((n_cols, tpw), jnp.int32),
            pltpu.VMEM((tpw, dim), jnp.float32),
            pltpu.VMEM((tpw, dim), jnp.float32),
            pltpu.VMEM((dim, tpad), jnp.float32),
            pltpu.VMEM((dim, tpad), jnp.float32),
            pltpu.SemaphoreType.DMA,
            pltpu.SemaphoreType.DMA,
            pltpu.SemaphoreType.DMA,
            pltpu.SemaphoreType.DMA,
        ],
        compiler_params=pltpu.CompilerParams(
            use_tc_tiling_on_sc=False, needs_layout_passes=False
        ),
    )
    def emb(xt_hbm, table_hbm, out_hbm, idx_t, rows0, rows1, tb0, tb1,
            gsem0, gsem1, wsem0, wsem1):
        wid = lax.axis_index("s") * info.num_cores + lax.axis_index("c")
        i0 = wid * tpw
        pltpu.sync_copy(xt_hbm.at[:, pl.ds(i0, tpw)], idx_t)
        rows = (rows0, rows1)
        tbufs = (tb0, tb1)
        gsems = (gsem0, gsem1)
        wsems = (wsem0, wsem1)

        def gather_descs(j, p):
            return [
                pltpu.make_async_copy(
                    table_hbm.at[idx_t.at[j, pl.ds(q * seg, seg)]],
                    rows[p].at[pl.ds(q * seg, seg)],
                    gsems[p],
                )
                for q in range(NSTREAM)
            ]

        def wb_desc(j, p):
            return pltpu.make_async_copy(
                tbufs[p].at[:, pl.ds(0, tpw)],
                out_hbm.at[j, :, pl.ds(i0, tpw)],
                wsems[p],
            )

        dlo = lax.iota(jnp.int32, lanes)
        dhi = dlo + lanes

        def transpose(p):
            src, dst = rows[p], tbufs[p]

            @plsc.parallel_loop(0, tpw, 8, unroll=2)
            def tbody(i):
                for t in range(8):
                    iv = jnp.full((lanes,), i + t, jnp.int32)
                    v0 = src[i + t, pl.ds(0, lanes)]
                    v1 = src[i + t, pl.ds(lanes, lanes)]
                    plsc.store_scatter(dst, [dlo, iv], v0)
                    plsc.store_scatter(dst, [dhi, iv], v1)

        for d in gather_descs(0, 0):
            d.start()
        for d in gather_descs(1, 1):
            d.start()

        def body(jj, carry):
            for p in range(2):
                j = 2 * jj + p
                for desc in gather_descs(j, p):
                    desc.wait()

                @pl.when(jj > 0)
                def _():
                    wb_desc(j - 2, p).wait()

                transpose(p)
                wb_desc(j, p).start()

                @pl.when(jj + 1 < npair)
                def _():
                    for desc in gather_descs(j + 2, p):
                        desc.start()

            return carry

        lax.fori_loop(0, npair, body, 0)
        wb_desc(n_cols - 2, 0).wait()
        wb_desc(n_cols - 1, 1).wait()

    return emb


def kernel(x, weights):
    b, s = x.shape
    dim = weights.shape[1]
    xt = jnp.transpose(x.astype(jnp.int32))
    out_t = _make(s, b, dim)(xt, weights)
    return jnp.transpose(out_t, (2, 0, 1))
